# baseline (speedup 1.0000x reference)
import functools

import jax
import jax.numpy as jnp
from jax import lax
from jax.experimental import pallas as pl
from jax.experimental.pallas import tpu as pltpu
from jax.experimental.pallas import tpu_sc as plsc

N_FIELDS_K = 26
EMBED_DIM_K = 32
BATCH_K = 16384

NUM_WORKERS = 32
ROWS_PER_WORKER = BATCH_K // NUM_WORKERS
NB = 32
NUM_CHUNKS = ROWS_PER_WORKER // NB
GB = 4
FIRES = NB // GB
NLANE = 16


def _sc_body(table_h, x_h, bias_h, out_h,
             idx_v, bias_v, buf0, buf1, sg0, sg1, ss0, ss1):
    wid = lax.axis_index("s") * 2 + lax.axis_index("c")
    base = wid * ROWS_PER_WORKER

    pltpu.sync_copy(x_h.at[pl.ds(pl.multiple_of(base, 8), ROWS_PER_WORKER)],
                    idx_v)
    pltpu.sync_copy(bias_h, bias_v)

    def fire(c, buf, sem):
        def one(i, _):
            pltpu.async_copy(
                table_h.at[idx_v.at[c * NB + i]],
                buf.at[i],
                sem,
            )
            return 0
        lax.fori_loop(0, NB, one, 0)

    def drain(buf, sem):
        pltpu.make_async_copy(out_h.at[pl.ds(0, NB)], buf, sem).wait()

    def add_bias(buf):
        def per_row(i, _):
            for f in range(N_FIELDS_K):
                for c16 in range(EMBED_DIM_K // NLANE):
                    sl = pl.ds(c16 * NLANE, NLANE)
                    buf[i, f, sl] = buf[i, f, sl] + bias_v[f, sl]
            return 0
        lax.fori_loop(0, NB, per_row, 0)

    def scatter(c, buf, sem):
        off = pl.multiple_of(base + c * NB, 8)
        pltpu.async_copy(buf, out_h.at[pl.ds(off, NB)], sem)

    def wait_scatter(c, buf, sem):
        off = pl.multiple_of(base + c * NB, 8)
        pltpu.make_async_copy(buf, out_h.at[pl.ds(off, NB)], sem).wait()

    fire(0, buf0, sg0)

    def pair_step(p, _):
        c0 = 2 * p
        c1 = c0 + 1
        @pl.when(p > 0)
        def _():
            wait_scatter(c1 - 2, buf1, ss1)
        fire(c1, buf1, sg1)
        drain(buf0, sg0)
        add_bias(buf0)
        scatter(c0, buf0, ss0)
        @pl.when(p < NUM_CHUNKS // 2 - 1)
        def _():
            wait_scatter(c0, buf0, ss0)
            fire(c0 + 2, buf0, sg0)
        drain(buf1, sg1)
        add_bias(buf1)
        scatter(c1, buf1, ss1)
        return 0

    lax.fori_loop(0, NUM_CHUNKS // 2, pair_step, 0)
    wait_scatter(NUM_CHUNKS - 2, buf0, ss0)
    wait_scatter(NUM_CHUNKS - 1, buf1, ss1)


@jax.jit
def kernel(X, table, bias):
    mesh = plsc.VectorSubcoreMesh(core_axis_name="c", subcore_axis_name="s")
    run = functools.partial(
        pl.kernel,
        mesh=mesh,
        out_type=jax.ShapeDtypeStruct((BATCH_K, N_FIELDS_K, EMBED_DIM_K),
                                      jnp.float32),
        scratch_types=[
            pltpu.VMEM((ROWS_PER_WORKER, N_FIELDS_K), jnp.int32),
            pltpu.VMEM((N_FIELDS_K, EMBED_DIM_K), jnp.float32),
            pltpu.VMEM((NB, N_FIELDS_K, EMBED_DIM_K), jnp.float32),
            pltpu.VMEM((NB, N_FIELDS_K, EMBED_DIM_K), jnp.float32),
            pltpu.SemaphoreType.DMA,
            pltpu.SemaphoreType.DMA,
            pltpu.SemaphoreType.DMA,
            pltpu.SemaphoreType.DMA,
        ],
        compiler_params=pltpu.CompilerParams(use_tc_tiling_on_sc=False),
    )(_sc_body)
    return run(table, X, bias)


# speedup vs baseline: 1.1099x; 1.1099x over previous
import functools

import jax
import jax.numpy as jnp
from jax import lax
from jax.experimental import pallas as pl
from jax.experimental.pallas import tpu as pltpu
from jax.experimental.pallas import tpu_sc as plsc

N_FIELDS_K = 26
EMBED_DIM_K = 32
BATCH_K = 16384

NUM_WORKERS = 32
RPW = BATCH_K // NUM_WORKERS
NLANE = 16


def _sc_body(table_h, x_h, bias_h, out_h,
             xblk, bias_v, idx0, idx1, stg0, stg1, trb0, trb1,
             sg0, sg1, ss0, ss1):
    wid = lax.axis_index("s") * 2 + lax.axis_index("c")
    b0 = wid * RPW

    pltpu.sync_copy(x_h.at[pl.ds(b0, RPW)], xblk)
    pltpu.sync_copy(bias_h, bias_v)

    iota = lax.iota(jnp.int32, NLANE)
    i_ts0 = iota // 8
    i_s0 = iota % 8
    i_ts1 = i_ts0 + 2

    def build_idx(f, idxb):
        def per_k(k, _):
            for g in range(8):
                rows = k * 128 + g * NLANE + iota
                cols = jnp.broadcast_to(f, (NLANE,))
                v = plsc.load_gather(xblk, [rows, cols])
                idxb[k, pl.ds(g * NLANE, NLANE)] = v
            return 0
        lax.fori_loop(0, 4, per_k, 0)

    def fire(idxb, stg, sem):
        for k in range(4):
            pltpu.async_copy(
                table_h.at[idxb.at[k]],
                stg.at[pl.ds(k * 128, 128)],
                sem,
            )

    def drain(stg, sem):
        pltpu.make_async_copy(table_h.at[pl.ds(0, RPW)], stg, sem).wait()

    def compute(f, stg, trb):
        bias_lo = bias_v[f, pl.ds(0, NLANE)]
        bias_hi = bias_v[f, pl.ds(NLANE, NLANE)]

        def per_row(l, _):
            tbl = jnp.broadcast_to(l // 128, (NLANE,))
            l128 = jnp.broadcast_to(l % 128, (NLANE,))
            lo = stg[l, pl.ds(0, NLANE)] + bias_lo
            hi = stg[l, pl.ds(NLANE, NLANE)] + bias_hi
            plsc.store_scatter(trb, [i_ts0, tbl, i_s0, l128], lo)
            plsc.store_scatter(trb, [i_ts1, tbl, i_s0, l128], hi)
            return 0
        lax.fori_loop(0, RPW, per_row, 0)

    def scatter_out(f, trb, sem):
        for ts in range(4):
            pltpu.async_copy(trb.at[ts], out_h.at[f, ts, pl.ds(wid * 4, 4)],
                             sem)

    def wait_scatter(f, trb, sem):
        for ts in range(4):
            pltpu.make_async_copy(trb.at[ts],
                                  out_h.at[f, ts, pl.ds(wid * 4, 4)],
                                  sem).wait()

    build_idx(0, idx0)
    fire(idx0, stg0, sg0)

    def pair_step(p, _):
        f0 = 2 * p
        f1 = f0 + 1

        @pl.when(p > 0)
        def _():
            wait_scatter(f1 - 2, trb1, ss1)
        build_idx(f1, idx1)
        fire(idx1, stg1, sg1)

        drain(stg0, sg0)
        compute(f0, stg0, trb0)
        scatter_out(f0, trb0, ss0)

        @pl.when(p < N_FIELDS_K // 2 - 1)
        def _():
            wait_scatter(f0, trb0, ss0)
            build_idx(f0 + 2, idx0)
            fire(idx0, stg0, sg0)

        drain(stg1, sg1)
        compute(f1, stg1, trb1)
        scatter_out(f1, trb1, ss1)
        return 0

    lax.fori_loop(0, N_FIELDS_K // 2, pair_step, 0)
    wait_scatter(N_FIELDS_K - 2, trb0, ss0)
    wait_scatter(N_FIELDS_K - 1, trb1, ss1)


@jax.jit
def kernel(X, table, bias):
    mesh = plsc.VectorSubcoreMesh(core_axis_name="c", subcore_axis_name="s")
    run = functools.partial(
        pl.kernel,
        mesh=mesh,
        out_type=jax.ShapeDtypeStruct(
            (N_FIELDS_K, 4, BATCH_K // 128, 8, 128), jnp.float32),
        scratch_types=[
            pltpu.VMEM((RPW, N_FIELDS_K), jnp.int32),
            pltpu.VMEM((N_FIELDS_K, EMBED_DIM_K), jnp.float32),
            pltpu.VMEM((4, 128), jnp.int32),
            pltpu.VMEM((4, 128), jnp.int32),
            pltpu.VMEM((RPW, EMBED_DIM_K), jnp.float32),
            pltpu.VMEM((RPW, EMBED_DIM_K), jnp.float32),
            pltpu.VMEM((4, 4, 8, 128), jnp.float32),
            pltpu.VMEM((4, 4, 8, 128), jnp.float32),
            pltpu.SemaphoreType.DMA,
            pltpu.SemaphoreType.DMA,
            pltpu.SemaphoreType.DMA,
            pltpu.SemaphoreType.DMA,
        ],
        compiler_params=pltpu.CompilerParams(use_tc_tiling_on_sc=False,
                                             needs_layout_passes=False),
    )(_sc_body)
    out5 = run(table, X, bias)
    return out5.transpose(2, 4, 0, 1, 3).reshape(BATCH_K, N_FIELDS_K,
                                                 EMBED_DIM_K)
